# baseline (speedup 1.0000x reference)
import jax
import jax.numpy as jnp
from jax.experimental import pallas as pl
from jax.experimental.pallas import tpu as pltpu

_CH = 8
_HID = 32
_KS = 5
_NL = 4
_NF = 4
_HC = _CH // 2
_PAD = (_KS - 1) // 2
_BB = 8
_G = 4


def _flows_kernel(x_ref, m_ref, gb_ref, e32_ref, e8_ref,
                  pre_ref, preb_ref, in_ref, rs_ref, rsb_ref,
                  skip_ref, skipb_ref, post_ref, postb_ref, ind1_ref,
                  out_ref):
    R = x_ref.shape[1]
    f32, bf16 = jnp.float32, jnp.bfloat16
    H = _HID
    half = jnp.bfloat16(0.5)
    z1 = jnp.zeros((1, H), bf16)
    z2 = jnp.zeros((2, H), bf16)

    for b in range(_BB):
        sf = x_ref[b]
        m4 = m_ref[b].astype(bf16)
        m32f = jnp.dot(m4, e32_ref[...], preferred_element_type=f32)
        m32 = m32f.astype(bf16)
        m8 = jnp.dot(m4, e8_ref[...], preferred_element_type=f32)

        for f in range(_NF):
            xcur = (jnp.dot(sf.astype(bf16), pre_ref[f],
                            preferred_element_type=f32)
                    + preb_ref[f])
            skip = None
            for i in range(_NL):
                xqm = xcur.astype(bf16) * m32
                cm2 = jnp.concatenate([z2, xqm[R - 2:, :3 * H]], axis=1)
                cm1 = jnp.concatenate([z1, xqm[R - 1:, :3 * H]], axis=1)
                cp1 = jnp.concatenate([xqm[:1, H:], z1], axis=1)
                cp2 = jnp.concatenate([xqm[:2, H:], z2], axis=1)
                tx = jnp.concatenate(
                    [jnp.concatenate([cm2, xqm[:R - 2]], axis=0),
                     jnp.concatenate([cm1, xqm[:R - 1]], axis=0),
                     xqm,
                     jnp.concatenate([xqm[1:], cp1], axis=0),
                     jnp.concatenate([xqm[2:], cp2], axis=0)],
                    axis=1)
                li = f * _NL + i
                z = (jnp.dot(tx, in_ref[li],
                             preferred_element_type=f32)
                     + gb_ref[b, li])
                tz = jnp.tanh(z).astype(bf16)
                acts = tz[:, :_G * H] * (tz[:, _G * H:] * half + half)
                if i < _NL - 1:
                    rsf = (jnp.dot(acts, rs_ref[f, i],
                                   preferred_element_type=f32)
                           + rsb_ref[f, i])
                    xcur = xcur + rsf[:, :_G * H]
                    sk = rsf[:, _G * H:]
                else:
                    sk = (jnp.dot(acts, skip_ref[f],
                                  preferred_element_type=f32)
                          + skipb_ref[f])
                skip = sk if skip is None else skip + sk
            mf = (jnp.dot(skip.astype(bf16), post_ref[f],
                          preferred_element_type=f32)
                  + postb_ref[f]) * m8
            blend = 1.0 + ind1_ref[f] * (m8 - 1.0)
            sf = sf * blend + mf
        out_ref[b] = sf.astype(out_ref.dtype)


def _bdiag(w):
    e = jnp.eye(_G, dtype=jnp.float32)
    out = w[..., None, :, None, :] * e[:, None, :, None]
    sh = w.shape
    return out.reshape(sh[:-2] + (_G * sh[-2], _G * sh[-1])).astype(jnp.bfloat16)


def _fold_t(a, last):
    B = a.shape[0]
    return (a.reshape(B, last, _G, -1).transpose(0, 3, 2, 1)
             .reshape(B, -1, _G * last))


def _tile_row(v):
    return jnp.tile(v[..., 0][..., None, :], (1,) * (v.ndim - 2) + (1, _G))


def kernel(x, x_mask, g, pre_w, pre_b, in_w, rs_w, rs_b, skip_w, skip_b,
           post_w, post_b, ind1, cond_w, cond_b, in_b, gate_scale):
    B, C, T = x.shape
    FL = _NF * _NL
    H = _HID
    R = T // _G

    g2 = g[:, :, 0]
    ga = jnp.einsum('bg,fog->fbo', g2, cond_w) + cond_b[:, None]
    ga = ga.reshape(_NF, B, _NL, 2 * H) + in_b[:, None]
    gb = jnp.transpose(ga, (1, 0, 2, 3)).reshape(B, FL, 2 * H)
    gb = gb * gate_scale
    gbrow = jnp.tile(gb.reshape(B, FL, 2, 1, H),
                     (1, 1, 1, _G, 1)).reshape(B, FL, 1, 2 * _G * H)

    w6 = in_w.reshape(_NF * _NL, 2, H, _KS, H)
    w6 = jnp.transpose(w6, (0, 3, 4, 1, 2))
    e = jnp.eye(_G, dtype=in_w.dtype)
    in_bd = (w6[:, :, None, :, :, None, :] * e[None, None, :, None, None, :, None]
             ).reshape(FL, _KS * _G * H, 2 * _G * H)

    pre_bd = _bdiag(jnp.swapaxes(pre_w.astype(jnp.float32), -1, -2))
    rs6 = jnp.transpose(rs_w.reshape(_NF, _NL - 1, 2, H, H),
                        (0, 1, 4, 2, 3))
    rs_bd = (rs6[:, :, None, :, :, None, :]
             * e[None, None, :, None, None, :, None]
             ).reshape(_NF, _NL - 1, _G * H, 2 * _G * H)
    skip_bd = _bdiag(jnp.swapaxes(skip_w.astype(jnp.float32), -1, -2))
    post_bd = _bdiag(jnp.swapaxes(post_w.astype(jnp.float32), -1, -2))

    preb_row = _tile_row(pre_b)
    rsb_row = jnp.tile(rs_b.reshape(_NF, _NL - 1, 2, 1, H),
                       (1, 1, 1, _G, 1)).reshape(_NF, _NL - 1, 1, 2 * _G * H)
    skipb_row = _tile_row(skip_b)
    postb_row = _tile_row(post_b)
    ind1_row = _tile_row(ind1)

    e32 = _bdiag(jnp.ones((1, H), jnp.float32))
    e8 = _bdiag(jnp.ones((1, C), jnp.float32))

    xf = _fold_t(x, C)
    mf4 = _fold_t(x_mask, 1)

    weights = [e32, e8, pre_bd, preb_row, in_bd, rs_bd, rsb_row,
               skip_bd, skipb_row, post_bd, postb_row, ind1_row]
    full = lambda a: pl.BlockSpec(a.shape, (lambda nd: (lambda p: (0,) * nd))(a.ndim))

    yf = pl.pallas_call(
        _flows_kernel,
        out_shape=jax.ShapeDtypeStruct((B, R, _G * C), x.dtype),
        grid=(B // _BB,),
        in_specs=[
            pl.BlockSpec((_BB, R, _G * C), lambda p: (p, 0, 0)),
            pl.BlockSpec((_BB, R, _G), lambda p: (p, 0, 0)),
            pl.BlockSpec((_BB, FL, 1, 2 * _G * H), lambda p: (p, 0, 0, 0)),
        ] + [full(w) for w in weights],
        out_specs=pl.BlockSpec((_BB, R, _G * C), lambda p: (p, 0, 0)),
        compiler_params=pltpu.CompilerParams(
            dimension_semantics=("parallel",)),
    )(xf, mf4, gbrow, *weights)

    y = yf.reshape(B, R, _G, C).transpose(0, 3, 2, 1).reshape(B, C, T)
    return y


# speedup vs baseline: 1.8163x; 1.8163x over previous
import jax
import jax.numpy as jnp
from jax.experimental import pallas as pl
from jax.experimental.pallas import tpu as pltpu

_CH = 8
_HID = 32
_KS = 5
_NL = 4
_NF = 4
_HC = _CH // 2
_PAD = (_KS - 1) // 2
_BB = 8


def _flows_kernel(x_ref, m_ref, gb_ref, pre_w_ref, pre_b_ref, in_w_ref,
                  rs_w_ref, rs_b_ref, skip_w_ref, skip_b_ref,
                  post_w_ref, post_b_ref, ind1_ref, out_ref):
    T = x_ref.shape[-1]
    f32, bf16 = jnp.float32, jnp.bfloat16
    H = _HID
    half = jnp.bfloat16(0.5)

    tpos = jax.lax.broadcasted_iota(jnp.int32, (1, T), 1)
    edge = {}
    for d in range(-_PAD, _PAD + 1):
        if d < 0:
            edge[d] = (tpos >= -d).astype(bf16)
        elif d > 0:
            edge[d] = (tpos < T - d).astype(bf16)

    for b in range(_BB):
        s = x_ref[b]
        mask = m_ref[b]
        for f in range(_NF):
            h = (jnp.dot(pre_w_ref[f], s.astype(bf16),
                         preferred_element_type=f32) + pre_b_ref[f]) * mask
            xcur = h
            skip = None
            for i in range(_NL):
                xq = xcur.astype(bf16)
                taps = []
                for j in range(_KS):
                    d = j - _PAD
                    if d == 0:
                        taps.append(xq)
                    else:
                        rot = jnp.concatenate([xq[:, d:], xq[:, :d]], axis=1)
                        taps.append(rot * edge[d])
                tcat = jnp.concatenate(taps, axis=0)
                z = (jnp.dot(in_w_ref[f, i], tcat,
                             preferred_element_type=f32)
                     + gb_ref[b, f * _NL + i])
                tz = jnp.tanh(z).astype(bf16)
                acts = tz[:H] * (tz[H:] * half + half)
                if i < _NL - 1:
                    rs = (jnp.dot(rs_w_ref[f, i], acts,
                                  preferred_element_type=f32) + rs_b_ref[f, i])
                    xcur = (xcur + rs[:H]) * mask
                    sk = rs[H:]
                else:
                    sk = (jnp.dot(skip_w_ref[f], acts,
                                  preferred_element_type=f32) + skip_b_ref[f])
                skip = sk if skip is None else skip + sk
            out = skip * mask
            mf = (jnp.dot(post_w_ref[f], out.astype(bf16),
                          preferred_element_type=f32) + post_b_ref[f]) * mask
            blend = 1.0 + ind1_ref[f] * (mask - 1.0)
            s = s * blend + mf
        out_ref[b] = s.astype(out_ref.dtype)


def kernel(x, x_mask, g, pre_w, pre_b, in_w, rs_w, rs_b, skip_w, skip_b,
           post_w, post_b, ind1, cond_w, cond_b, in_b, gate_scale):
    B, C, T = x.shape
    FL = _NF * _NL

    g2 = g[:, :, 0]
    ga = jnp.einsum('bg,fog->fbo', g2, cond_w) + cond_b[:, None]
    ga = ga.reshape(_NF, B, _NL, 2 * _HID) + in_b[:, None]
    gb = jnp.transpose(ga, (1, 0, 2, 3)).reshape(B, FL, 2 * _HID)
    gb = (gb * gate_scale)[..., None]

    weights = [pre_w, pre_b, in_w, rs_w, rs_b, skip_w, skip_b,
               post_w, post_b, ind1]
    full = lambda a: pl.BlockSpec(a.shape, (lambda nd: (lambda p: (0,) * nd))(a.ndim))

    y = pl.pallas_call(
        _flows_kernel,
        out_shape=jax.ShapeDtypeStruct((B, C, T), x.dtype),
        grid=(B // _BB,),
        in_specs=[
            pl.BlockSpec((_BB, C, T), lambda p: (p, 0, 0)),
            pl.BlockSpec((_BB, 1, T), lambda p: (p, 0, 0)),
            pl.BlockSpec((_BB, FL, 2 * _HID, 1), lambda p: (p, 0, 0, 0)),
        ] + [full(w) for w in weights],
        out_specs=pl.BlockSpec((_BB, C, T), lambda p: (p, 0, 0)),
        compiler_params=pltpu.CompilerParams(
            dimension_semantics=("parallel",)),
    )(x, x_mask, gb, *weights)
    return y


# speedup vs baseline: 1.8925x; 1.0419x over previous
import jax
import jax.numpy as jnp
from jax.experimental import pallas as pl
from jax.experimental.pallas import tpu as pltpu

_CH = 8
_HID = 32
_KS = 5
_NL = 4
_NF = 4
_HC = _CH // 2
_PAD = (_KS - 1) // 2
_BB = 16


def _flows_kernel(x_ref, m_ref, gb_ref, pre_w_ref, pre_b_ref, in_w_ref,
                  rs_w_ref, rs_b_ref, skip_w_ref, skip_b_ref,
                  post_w_ref, post_b_ref, ind1_ref, out_ref):
    T = x_ref.shape[-1]
    f32, bf16 = jnp.float32, jnp.bfloat16
    H = _HID
    half = jnp.bfloat16(0.5)

    tpos = jax.lax.broadcasted_iota(jnp.int32, (1, T), 1)
    edge = {}
    for d in range(-_PAD, _PAD + 1):
        if d < 0:
            edge[d] = (tpos >= -d).astype(bf16)
        elif d > 0:
            edge[d] = (tpos < T - d).astype(bf16)

    for b in range(_BB):
        s = x_ref[b]
        mask = m_ref[b]
        for f in range(_NF):
            h = (jnp.dot(pre_w_ref[f], s.astype(bf16),
                         preferred_element_type=f32) + pre_b_ref[f]) * mask
            xcur = h
            skip = None
            for i in range(_NL):
                xq = xcur.astype(bf16)
                taps = []
                for j in range(_KS):
                    d = j - _PAD
                    if d == 0:
                        taps.append(xq)
                    else:
                        rot = jnp.concatenate([xq[:, d:], xq[:, :d]], axis=1)
                        taps.append(rot * edge[d])
                tcat = jnp.concatenate(taps, axis=0)
                z = (jnp.dot(in_w_ref[f, i], tcat,
                             preferred_element_type=f32)
                     + gb_ref[b, f * _NL + i])
                tz = jnp.tanh(z).astype(bf16)
                acts = tz[:H] * (tz[H:] * half + half)
                if i < _NL - 1:
                    rs = (jnp.dot(rs_w_ref[f, i], acts,
                                  preferred_element_type=f32) + rs_b_ref[f, i])
                    xcur = (xcur + rs[:H]) * mask
                    sk = rs[H:]
                else:
                    sk = (jnp.dot(skip_w_ref[f], acts,
                                  preferred_element_type=f32) + skip_b_ref[f])
                skip = sk if skip is None else skip + sk
            out = skip * mask
            mf = (jnp.dot(post_w_ref[f], out.astype(bf16),
                          preferred_element_type=f32) + post_b_ref[f]) * mask
            blend = 1.0 + ind1_ref[f] * (mask - 1.0)
            s = s * blend + mf
        out_ref[b] = s.astype(out_ref.dtype)


def kernel(x, x_mask, g, pre_w, pre_b, in_w, rs_w, rs_b, skip_w, skip_b,
           post_w, post_b, ind1, cond_w, cond_b, in_b, gate_scale):
    B, C, T = x.shape
    FL = _NF * _NL

    g2 = g[:, :, 0]
    ga = jnp.einsum('bg,fog->fbo', g2, cond_w) + cond_b[:, None]
    ga = ga.reshape(_NF, B, _NL, 2 * _HID) + in_b[:, None]
    gb = jnp.transpose(ga, (1, 0, 2, 3)).reshape(B, FL, 2 * _HID)
    gb = (gb * gate_scale)[..., None]

    weights = [pre_w, pre_b, in_w, rs_w, rs_b, skip_w, skip_b,
               post_w, post_b, ind1]
    full = lambda a: pl.BlockSpec(a.shape, (lambda nd: (lambda p: (0,) * nd))(a.ndim))

    y = pl.pallas_call(
        _flows_kernel,
        out_shape=jax.ShapeDtypeStruct((B, C, T), x.dtype),
        grid=(B // _BB,),
        in_specs=[
            pl.BlockSpec((_BB, C, T), lambda p: (p, 0, 0)),
            pl.BlockSpec((_BB, 1, T), lambda p: (p, 0, 0)),
            pl.BlockSpec((_BB, FL, 2 * _HID, 1), lambda p: (p, 0, 0, 0)),
        ] + [full(w) for w in weights],
        out_specs=pl.BlockSpec((_BB, C, T), lambda p: (p, 0, 0)),
        compiler_params=pltpu.CompilerParams(
            dimension_semantics=("parallel",)),
    )(x, x_mask, gb, *weights)
    return y
